# Initial kernel scaffold; baseline (speedup 1.0000x reference)
import jax
import jax.numpy as jnp
from jax.experimental import pallas as pl


def kernel(features, coords):
    raise NotImplementedError("write your pallas kernel here")


# speedup vs baseline: 3.0947x; 3.0947x over previous
import functools

import jax
import jax.numpy as jnp
from jax import lax
from jax.experimental import pallas as pl
from jax.experimental.pallas import tpu as pltpu
from jax.experimental.pallas import tpu_sc as plsc

_R = 32
_NV = _R * _R * _R
_B, _C, _N = 8, 3, 65536
_FC = 32
_GROUPS = 4
_CPG = _FC // _GROUPS
_CCH = 2048
_FCH = 4096
_DCH = 2048


def _make_sc_kernel():
  mesh = plsc.VectorSubcoreMesh(core_axis_name="c", subcore_axis_name="s")

  @functools.partial(
      pl.kernel,
      out_type=(
          jax.ShapeDtypeStruct((_B, _FC, _NV), jnp.float32),
          jax.ShapeDtypeStruct((_B, 3, _N), jnp.int32),
      ),
      mesh=mesh,
      scratch_types=[
          pltpu.VMEM((_N,), jnp.int32),
          pltpu.VMEM((_NV,), jnp.float32),
          pltpu.VMEM((2, _FCH), jnp.float32),
          pltpu.VMEM((3, _CCH), jnp.float32),
          pltpu.VMEM((3, _CCH), jnp.int32),
          pltpu.VMEM_SHARED((_GROUPS, _NV), jnp.float32),
      ],
      compiler_params=pltpu.CompilerParams(needs_layout_passes=False),
  )
  def vox_kernel(f_hbm, c_hbm, out_hbm, vox_hbm,
                 idx_v, acc_v, fbuf, cbuf, vbuf, inv_s):
    cid = lax.axis_index("c")
    sid = lax.axis_index("s")
    wid = cid * 16 + sid
    b = wid // _GROUPS
    g = wid % _GROUPS
    bl = b % _GROUPS

    half = jnp.float32(0.5)
    one = jnp.float32(1.0)
    zero16 = jnp.zeros((16,), jnp.float32)
    ones16 = jnp.ones((16,), jnp.float32)

    def p1_chunk(kc, _):
      base = kc * _CCH
      pltpu.sync_copy(c_hbm.at[b, :, pl.ds(base, _CCH)], cbuf)

      def p1_vec(i, _):
        off = i * 16

        def axis_round(a):
          t = jnp.clip(cbuf[a, pl.ds(off, 16)] * _R, 0.0, _R - 1.0)
          i0 = t.astype(jnp.int32)
          frac = t - i0.astype(jnp.float32)
          up = jnp.where(frac > half, 1, 0) + jnp.where(
              jnp.logical_and(frac == half, (i0 & 1) == 1), 1, 0)
          return i0 + up

        vx = axis_round(0)
        vy = axis_round(1)
        vz = axis_round(2)
        vbuf[0, pl.ds(off, 16)] = vx
        vbuf[1, pl.ds(off, 16)] = vy
        vbuf[2, pl.ds(off, 16)] = vz
        idx_v[pl.ds(base + off, 16)] = vx * (_R * _R) + vy * _R + vz
        return 0

      lax.fori_loop(0, _CCH // 16, p1_vec, 0)

      @pl.when(g == 0)
      def _():
        pltpu.sync_copy(vbuf, vox_hbm.at[b, :, pl.ds(base, _CCH)])

      return 0

    lax.fori_loop(0, _N // _CCH, p1_chunk, 0)

    @pl.when(g == _GROUPS - 1)
    def _counts():
      def zloop(j, _):
        acc_v[pl.ds(j * 16, 16)] = zero16
        return 0

      lax.fori_loop(0, _NV // 16, zloop, 0)

      def sloop(i, _):
        iv = idx_v[pl.ds(i * 16, 16)]
        plsc.addupdate_scatter(acc_v, [iv], ones16)
        return 0

      lax.fori_loop(0, _N // 16, sloop, 0)

      def iloop(j, _):
        v = acc_v[pl.ds(j * 16, 16)]
        acc_v[pl.ds(j * 16, 16)] = one / jnp.maximum(v, one)
        return 0

      lax.fori_loop(0, _NV // 16, iloop, 0)
      pltpu.sync_copy(acc_v, inv_s.at[bl])

    plsc.subcore_barrier()

    for cc in range(_CPG):
      ch = g * _CPG + cc

      def zloop2(j, _):
        acc_v[pl.ds(j * 16, 16)] = zero16
        return 0

      lax.fori_loop(0, _NV // 16, zloop2, 0)

      def f_chunk(kc, _):
        fbase = kc * _FCH
        pltpu.sync_copy(f_hbm.at[b, ch, pl.ds(fbase, _FCH)], fbuf.at[0])

        def svec(i, _):
          iv = idx_v[pl.ds(fbase + i * 16, 16)]
          fv = fbuf[0, pl.ds(i * 16, 16)]
          plsc.addupdate_scatter(acc_v, [iv], fv)
          return 0

        lax.fori_loop(0, _FCH // 16, svec, 0)
        return 0

      lax.fori_loop(0, _N // _FCH, f_chunk, 0)

      def d_chunk(m, _):
        dbase = m * _DCH
        pltpu.sync_copy(inv_s.at[bl, pl.ds(dbase, _DCH)],
                        fbuf.at[1, pl.ds(0, _DCH)])

        def dvec(j, _):
          a = acc_v[pl.ds(dbase + j * 16, 16)]
          fbuf[0, pl.ds(j * 16, 16)] = a * fbuf[1, pl.ds(j * 16, 16)]
          return 0

        lax.fori_loop(0, _DCH // 16, dvec, 0)
        pltpu.sync_copy(fbuf.at[0, pl.ds(0, _DCH)],
                        out_hbm.at[b, ch, pl.ds(dbase, _DCH)])
        return 0

      lax.fori_loop(0, _NV // _DCH, d_chunk, 0)

  return vox_kernel


_vox_kernel = _make_sc_kernel()


@jax.jit
def kernel(features, coords):
  out_flat, vox_coords = _vox_kernel(features, coords)
  out = out_flat.reshape(_B, _FC, _R, _R, _R)
  return out, vox_coords


# speedup vs baseline: 3.8772x; 1.2528x over previous
import functools

import jax
import jax.numpy as jnp
from jax import lax
from jax.experimental import pallas as pl
from jax.experimental.pallas import tpu as pltpu
from jax.experimental.pallas import tpu_sc as plsc

_R = 32
_NV = _R * _R * _R
_B, _N = 8, 65536
_FC = 32
_GROUPS = 4
_CPG = _FC // _GROUPS
_CCH = 1024
_FCH = 4096
_NCH = _N // _FCH
_DCH = 2048


def _make_sc_kernel():
  mesh = plsc.VectorSubcoreMesh(core_axis_name="c", subcore_axis_name="s")

  @functools.partial(
      pl.kernel,
      out_type=(
          jax.ShapeDtypeStruct((_B, _FC, _NV), jnp.float32),
          jax.ShapeDtypeStruct((_B, 3, _N), jnp.int32),
      ),
      mesh=mesh,
      scratch_types=[
          pltpu.VMEM((_N,), jnp.int32),
          pltpu.VMEM((_NV,), jnp.float32),
          pltpu.VMEM((2, _FCH), jnp.float32),
          pltpu.VMEM((2, _DCH), jnp.float32),
          pltpu.VMEM((3, _CCH), jnp.float32),
          pltpu.VMEM((3, _CCH), jnp.int32),
          pltpu.VMEM_SHARED((_GROUPS, _NV), jnp.float32),
          pltpu.SemaphoreType.DMA((2,)),
      ],
      compiler_params=pltpu.CompilerParams(needs_layout_passes=False),
  )
  def vox_kernel(f_hbm, c_hbm, out_hbm, vox_hbm,
                 idx_v, acc_v, fbuf, dbuf, cbuf, vbuf, inv_s, fsem):
    cid = lax.axis_index("c")
    sid = lax.axis_index("s")
    wid = cid * 16 + sid
    b = wid // _GROUPS
    g = wid % _GROUPS
    bl = b % _GROUPS

    half = jnp.float32(0.5)
    one = jnp.float32(1.0)
    zero16 = jnp.zeros((16,), jnp.float32)
    ones16 = jnp.ones((16,), jnp.float32)

    def p1_chunk(kc, _):
      base = kc * _CCH
      pltpu.sync_copy(c_hbm.at[b, :, pl.ds(base, _CCH)], cbuf)

      def p1_vec(i, _):
        for u in range(2):
          off = i * 32 + u * 16

          def axis_round(a):
            t = jnp.clip(cbuf[a, pl.ds(off, 16)] * _R, 0.0, _R - 1.0)
            i0 = t.astype(jnp.int32)
            frac = t - i0.astype(jnp.float32)
            up = jnp.where(frac > half, 1, 0) + jnp.where(
                jnp.logical_and(frac == half, (i0 & 1) == 1), 1, 0)
            return i0 + up

          vx = axis_round(0)
          vy = axis_round(1)
          vz = axis_round(2)
          vbuf[0, pl.ds(off, 16)] = vx
          vbuf[1, pl.ds(off, 16)] = vy
          vbuf[2, pl.ds(off, 16)] = vz
          idx_v[pl.ds(base + off, 16)] = vx * (_R * _R) + vy * _R + vz
        return 0

      lax.fori_loop(0, _CCH // 32, p1_vec, 0)

      @pl.when(g == 0)
      def _():
        pltpu.sync_copy(vbuf, vox_hbm.at[b, :, pl.ds(base, _CCH)])

      return 0

    lax.fori_loop(0, _N // _CCH, p1_chunk, 0)

    @pl.when(g == _GROUPS - 1)
    def _counts():
      def zloop(j, _):
        for u in range(8):
          acc_v[pl.ds((j * 8 + u) * 16, 16)] = zero16
        return 0

      lax.fori_loop(0, _NV // 128, zloop, 0)

      def sloop(i, _):
        for u in range(8):
          iv = idx_v[pl.ds((i * 8 + u) * 16, 16)]
          plsc.addupdate_scatter(acc_v, [iv], ones16)
        return 0

      lax.fori_loop(0, _N // 128, sloop, 0)

      def iloop(j, _):
        for u in range(4):
          o = (j * 4 + u) * 16
          v = acc_v[pl.ds(o, 16)]
          acc_v[pl.ds(o, 16)] = one / jnp.maximum(v, one)
        return 0

      lax.fori_loop(0, _NV // 64, iloop, 0)
      pltpu.sync_copy(acc_v, inv_s.at[bl])

    @pl.when(g != _GROUPS - 1)
    def _zero0():
      def zloop2(j, _):
        for u in range(8):
          acc_v[pl.ds((j * 8 + u) * 16, 16)] = zero16
        return 0

      lax.fori_loop(0, _NV // 128, zloop2, 0)

    plsc.subcore_barrier()

    @pl.when(g == _GROUPS - 1)
    def _zero3():
      def zloop3(j, _):
        for u in range(8):
          acc_v[pl.ds((j * 8 + u) * 16, 16)] = zero16
        return 0

      lax.fori_loop(0, _NV // 128, zloop3, 0)

    ch0 = g * _CPG
    for slot in range(2):
      pltpu.async_copy(f_hbm.at[b, ch0, pl.ds(slot * _FCH, _FCH)],
                       fbuf.at[slot], fsem.at[slot])

    for cc in range(_CPG):
      ch = ch0 + cc

      def f_pair(kp, _):
        for slot in range(2):
          kc = kp * 2 + slot
          fbase = kc * _FCH
          pltpu.make_async_copy(f_hbm.at[b, ch, pl.ds(fbase, _FCH)],
                                fbuf.at[slot], fsem.at[slot]).wait()

          def svec(i, _):
            for u in range(8):
              o = i * 128 + u * 16
              iv = idx_v[pl.ds(fbase + o, 16)]
              fv = fbuf[slot, pl.ds(o, 16)]
              plsc.addupdate_scatter(acc_v, [iv], fv)
            return 0

          lax.fori_loop(0, _FCH // 128, svec, 0)

          nxt = fbase + 2 * _FCH

          @pl.when(nxt < _N)
          def _():
            pltpu.async_copy(f_hbm.at[b, ch, pl.ds(nxt, _FCH)],
                             fbuf.at[slot], fsem.at[slot])
        return 0

      lax.fori_loop(0, _NCH // 2, f_pair, 0)

      if cc + 1 < _CPG:
        for slot in range(2):
          pltpu.async_copy(f_hbm.at[b, ch + 1, pl.ds(slot * _FCH, _FCH)],
                           fbuf.at[slot], fsem.at[slot])

      def d_chunk(m, _):
        dbase = m * _DCH
        pltpu.sync_copy(inv_s.at[bl, pl.ds(dbase, _DCH)], dbuf.at[1])

        def dvec(j, _):
          for u in range(8):
            o = (j * 8 + u) * 16
            a = acc_v[pl.ds(dbase + o, 16)]
            dbuf[0, pl.ds(o, 16)] = a * dbuf[1, pl.ds(o, 16)]
            acc_v[pl.ds(dbase + o, 16)] = zero16
          return 0

        lax.fori_loop(0, _DCH // 128, dvec, 0)
        pltpu.sync_copy(dbuf.at[0], out_hbm.at[b, ch, pl.ds(dbase, _DCH)])
        return 0

      lax.fori_loop(0, _NV // _DCH, d_chunk, 0)

  return vox_kernel


_vox_kernel = _make_sc_kernel()


@jax.jit
def kernel(features, coords):
  out_flat, vox_coords = _vox_kernel(features, coords)
  out = out_flat.reshape(_B, _FC, _R, _R, _R)
  return out, vox_coords


# speedup vs baseline: 5.1568x; 1.3300x over previous
import functools

import jax
import jax.numpy as jnp
from jax import lax
from jax.experimental import pallas as pl
from jax.experimental.pallas import tpu as pltpu
from jax.experimental.pallas import tpu_sc as plsc

_R = 32
_NV = _R * _R * _R
_B, _N = 8, 65536
_FC = 32
_GROUPS = 4
_CPG = _FC // _GROUPS
_CCH = 1024
_NCC = _N // _CCH
_FCH = 2048
_NCH = _N // _FCH
_DCH = 2048
_NDC = _NV // _DCH


def _make_sc_kernel():
  mesh = plsc.VectorSubcoreMesh(core_axis_name="c", subcore_axis_name="s")

  @functools.partial(
      pl.kernel,
      out_type=(
          jax.ShapeDtypeStruct((_B, _FC, _NV), jnp.float32),
          jax.ShapeDtypeStruct((_B, 3, _N), jnp.int32),
      ),
      mesh=mesh,
      scratch_types=[
          pltpu.VMEM((_N,), jnp.int16),
          pltpu.VMEM((_NV,), jnp.float32),
          pltpu.VMEM((_NV,), jnp.float32),
          pltpu.VMEM((2, _FCH), jnp.float32),
          pltpu.VMEM((2, _DCH), jnp.float32),
          pltpu.VMEM((2, 3, _CCH), jnp.float32),
          pltpu.VMEM((2, 3, _CCH), jnp.int32),
          pltpu.VMEM_SHARED((_GROUPS, _NV), jnp.float32),
          pltpu.SemaphoreType.DMA((8,)),
      ],
      compiler_params=pltpu.CompilerParams(needs_layout_passes=False),
  )
  def vox_kernel(f_hbm, c_hbm, out_hbm, vox_hbm,
                 idx_v, acc_v, inv_v, fbuf, dbuf, cbuf, vbuf, inv_s, sems):
    cid = lax.axis_index("c")
    sid = lax.axis_index("s")
    wid = cid * 16 + sid
    b = wid // _GROUPS
    g = wid % _GROUPS
    bl = b % _GROUPS

    half = jnp.float32(0.5)
    one = jnp.float32(1.0)
    zero16 = jnp.zeros((16,), jnp.float32)
    ones16 = jnp.ones((16,), jnp.float32)

    def axis_round(vals):
      t = jnp.clip(vals * _R, 0.0, _R - 1.0)
      i0 = t.astype(jnp.int32)
      frac = t - i0.astype(jnp.float32)
      up = jnp.where(frac > half, 1, 0) + jnp.where(
          jnp.logical_and(frac == half, (i0 & 1) == 1), 1, 0)
      return i0 + up

    for slot in range(2):
      pltpu.async_copy(c_hbm.at[b, :, pl.ds(slot * _CCH, _CCH)],
                       cbuf.at[slot], sems.at[4 + slot])

    def p1_pair(kp, _):
      for slot in range(2):
        kc = kp * 2 + slot
        base = kc * _CCH
        pltpu.make_async_copy(c_hbm.at[b, :, pl.ds(base, _CCH)],
                              cbuf.at[slot], sems.at[4 + slot]).wait()

        @pl.when(jnp.logical_and(g == 0, kp >= 1))
        def _():
          pltpu.make_async_copy(vbuf.at[slot],
                                vox_hbm.at[b, :, pl.ds(base, _CCH)],
                                sems.at[6 + slot]).wait()

        @pl.when(g == 0)
        def _():
          def p1v_full(i, _):
            off = i * 32
            flats = []
            for u in range(2):
              o = off + u * 16
              vx = axis_round(cbuf[slot, 0, pl.ds(o, 16)])
              vy = axis_round(cbuf[slot, 1, pl.ds(o, 16)])
              vz = axis_round(cbuf[slot, 2, pl.ds(o, 16)])
              vbuf[slot, 0, pl.ds(o, 16)] = vx
              vbuf[slot, 1, pl.ds(o, 16)] = vy
              vbuf[slot, 2, pl.ds(o, 16)] = vz
              flats.append(vx * (_R * _R) + vy * _R + vz)
            packed = plsc.pack(flats[0], flats[1],
                               format=plsc.PackFormat.INTERLEAVED)
            idx_v[pl.ds(base + off, 32)] = packed
            return 0

          lax.fori_loop(0, _CCH // 32, p1v_full, 0)
          pltpu.async_copy(vbuf.at[slot],
                           vox_hbm.at[b, :, pl.ds(base, _CCH)],
                           sems.at[6 + slot])

        @pl.when(g != 0)
        def _():
          def p1v_idx(i, _):
            off = i * 32
            flats = []
            for u in range(2):
              o = off + u * 16
              vx = axis_round(cbuf[slot, 0, pl.ds(o, 16)])
              vy = axis_round(cbuf[slot, 1, pl.ds(o, 16)])
              vz = axis_round(cbuf[slot, 2, pl.ds(o, 16)])
              flats.append(vx * (_R * _R) + vy * _R + vz)
            packed = plsc.pack(flats[0], flats[1],
                               format=plsc.PackFormat.INTERLEAVED)
            idx_v[pl.ds(base + off, 32)] = packed
            return 0

          lax.fori_loop(0, _CCH // 32, p1v_idx, 0)

        nxt = base + 2 * _CCH

        @pl.when(nxt < _N)
        def _():
          pltpu.async_copy(c_hbm.at[b, :, pl.ds(nxt, _CCH)],
                           cbuf.at[slot], sems.at[4 + slot])
      return 0

    lax.fori_loop(0, _NCC // 2, p1_pair, 0)

    @pl.when(g == 0)
    def _():
      for slot in range(2):
        pltpu.make_async_copy(vbuf.at[slot],
                              vox_hbm.at[b, :, pl.ds(slot * _CCH, _CCH)],
                              sems.at[6 + slot]).wait()

    def zero_acc():
      def zloop(j, _):
        for u in range(8):
          acc_v[pl.ds((j * 8 + u) * 16, 16)] = zero16
        return 0

      lax.fori_loop(0, _NV // 128, zloop, 0)

    @pl.when(g == _GROUPS - 1)
    def _counts():
      zero_acc()

      def sloop(i, _):
        for u in range(4):
          o = (i * 4 + u) * 32
          iv0, iv1 = plsc.unpack(idx_v[pl.ds(o, 32)],
                                 format=plsc.PackFormat.INTERLEAVED)
          plsc.addupdate_scatter(acc_v, [iv0], ones16)
          plsc.addupdate_scatter(acc_v, [iv1], ones16)
        return 0

      lax.fori_loop(0, _N // 128, sloop, 0)

      def iloop(j, _):
        for u in range(4):
          o = (j * 4 + u) * 16
          v = acc_v[pl.ds(o, 16)]
          acc_v[pl.ds(o, 16)] = one / jnp.maximum(v, one)
        return 0

      lax.fori_loop(0, _NV // 64, iloop, 0)
      pltpu.sync_copy(acc_v, inv_s.at[bl])

    @pl.when(g != _GROUPS - 1)
    def _zero0():
      zero_acc()

    plsc.subcore_barrier()

    pltpu.sync_copy(inv_s.at[bl], inv_v)

    @pl.when(g == _GROUPS - 1)
    def _zero3():
      zero_acc()

    ch0 = g * _CPG
    for slot in range(2):
      pltpu.async_copy(f_hbm.at[b, ch0, pl.ds(slot * _FCH, _FCH)],
                       fbuf.at[slot], sems.at[slot])

    for cc in range(_CPG):
      ch = ch0 + cc

      def f_pair(kp, _):
        for slot in range(2):
          kc = kp * 2 + slot
          fbase = kc * _FCH
          pltpu.make_async_copy(f_hbm.at[b, ch, pl.ds(fbase, _FCH)],
                                fbuf.at[slot], sems.at[slot]).wait()

          def svec(i, _):
            for u in range(4):
              o = i * 128 + u * 32
              iv0, iv1 = plsc.unpack(idx_v[pl.ds(fbase + o, 32)],
                                     format=plsc.PackFormat.INTERLEAVED)
              fv0 = fbuf[slot, pl.ds(o, 16)]
              fv1 = fbuf[slot, pl.ds(o + 16, 16)]
              plsc.addupdate_scatter(acc_v, [iv0], fv0)
              plsc.addupdate_scatter(acc_v, [iv1], fv1)
            return 0

          lax.fori_loop(0, _FCH // 128, svec, 0)

          nxt = fbase + 2 * _FCH

          @pl.when(nxt < _N)
          def _():
            pltpu.async_copy(f_hbm.at[b, ch, pl.ds(nxt, _FCH)],
                             fbuf.at[slot], sems.at[slot])
        return 0

      lax.fori_loop(0, _NCH // 2, f_pair, 0)

      if cc + 1 < _CPG:
        for slot in range(2):
          pltpu.async_copy(f_hbm.at[b, ch + 1, pl.ds(slot * _FCH, _FCH)],
                           fbuf.at[slot], sems.at[slot])

      def d_pair(mp, _):
        for slot in range(2):
          m = mp * 2 + slot
          dbase = m * _DCH

          @pl.when(mp >= 1)
          def _():
            pltpu.make_async_copy(dbuf.at[slot],
                                  out_hbm.at[b, ch, pl.ds(dbase, _DCH)],
                                  sems.at[2 + slot]).wait()

          def dvec(j, _):
            for u in range(8):
              o = (j * 8 + u) * 16
              a = acc_v[pl.ds(dbase + o, 16)]
              dbuf[slot, pl.ds(o, 16)] = a * inv_v[pl.ds(dbase + o, 16)]
              acc_v[pl.ds(dbase + o, 16)] = zero16
            return 0

          lax.fori_loop(0, _DCH // 128, dvec, 0)
          pltpu.async_copy(dbuf.at[slot],
                           out_hbm.at[b, ch, pl.ds(dbase, _DCH)],
                           sems.at[2 + slot])
        return 0

      lax.fori_loop(0, _NDC // 2, d_pair, 0)

      for slot in range(2):
        pltpu.make_async_copy(dbuf.at[slot],
                              out_hbm.at[b, ch, pl.ds(slot * _DCH, _DCH)],
                              sems.at[2 + slot]).wait()

  return vox_kernel


_vox_kernel = _make_sc_kernel()


@jax.jit
def kernel(features, coords):
  out_flat, vox_coords = _vox_kernel(features, coords)
  out = out_flat.reshape(_B, _FC, _R, _R, _R)
  return out, vox_coords


# speedup vs baseline: 5.1679x; 1.0022x over previous
import functools

import jax
import jax.numpy as jnp
from jax import lax
from jax.experimental import pallas as pl
from jax.experimental.pallas import tpu as pltpu
from jax.experimental.pallas import tpu_sc as plsc

_R = 32
_NV = _R * _R * _R
_B, _N = 8, 65536
_FC = 32
_GROUPS = 4
_CPG = _FC // _GROUPS
_CCH = 1024
_NCC = _N // _CCH
_FCH = 2048
_NCH = _N // _FCH
_DCH = 2048
_NDC = _NV // _DCH


def _make_sc_kernel():
  mesh = plsc.VectorSubcoreMesh(core_axis_name="c", subcore_axis_name="s")

  @functools.partial(
      pl.kernel,
      out_type=(
          jax.ShapeDtypeStruct((_B, _FC, _NV), jnp.float32),
          jax.ShapeDtypeStruct((_B, 3, _N), jnp.int32),
      ),
      mesh=mesh,
      scratch_types=[
          pltpu.VMEM((_N // 2,), jnp.int32),
          pltpu.VMEM((_NV,), jnp.float32),
          pltpu.VMEM((_NV,), jnp.float32),
          pltpu.VMEM((2, _FCH), jnp.float32),
          pltpu.VMEM((2, _DCH), jnp.float32),
          pltpu.VMEM((2, 3, _CCH), jnp.float32),
          pltpu.VMEM((2, 3, _CCH), jnp.int32),
          pltpu.VMEM_SHARED((_GROUPS, _NV), jnp.float32),
          pltpu.SemaphoreType.DMA((8,)),
      ],
      compiler_params=pltpu.CompilerParams(needs_layout_passes=False),
  )
  def vox_kernel(f_hbm, c_hbm, out_hbm, vox_hbm,
                 idx_v, acc_v, inv_v, fbuf, dbuf, cbuf, vbuf, inv_s, sems):
    cid = lax.axis_index("c")
    sid = lax.axis_index("s")
    wid = cid * 16 + sid
    b = wid // _GROUPS
    g = wid % _GROUPS
    bl = b % _GROUPS

    half = jnp.float32(0.5)
    one = jnp.float32(1.0)
    zero16 = jnp.zeros((16,), jnp.float32)
    ones16 = jnp.ones((16,), jnp.float32)

    def axis_round(vals):
      t = jnp.clip(vals * _R, 0.0, _R - 1.0)
      i0 = t.astype(jnp.int32)
      frac = t - i0.astype(jnp.float32)
      up = jnp.where(frac > half, 1, 0) + jnp.where(
          jnp.logical_and(frac == half, (i0 & 1) == 1), 1, 0)
      return i0 + up

    for slot in range(2):
      pltpu.async_copy(c_hbm.at[b, :, pl.ds(slot * _CCH, _CCH)],
                       cbuf.at[slot], sems.at[4 + slot])

    def p1_pair(kp, _):
      for slot in range(2):
        kc = kp * 2 + slot
        base = kc * _CCH
        pltpu.make_async_copy(c_hbm.at[b, :, pl.ds(base, _CCH)],
                              cbuf.at[slot], sems.at[4 + slot]).wait()

        @pl.when(jnp.logical_and(g == 0, kp >= 1))
        def _():
          pltpu.make_async_copy(vbuf.at[slot],
                                vox_hbm.at[b, :, pl.ds(base, _CCH)],
                                sems.at[6 + slot]).wait()

        @pl.when(g == 0)
        def _():
          def p1v_full(i, _):
            off = i * 32
            flats = []
            for u in range(2):
              o = off + u * 16
              vx = axis_round(cbuf[slot, 0, pl.ds(o, 16)])
              vy = axis_round(cbuf[slot, 1, pl.ds(o, 16)])
              vz = axis_round(cbuf[slot, 2, pl.ds(o, 16)])
              vbuf[slot, 0, pl.ds(o, 16)] = vx
              vbuf[slot, 1, pl.ds(o, 16)] = vy
              vbuf[slot, 2, pl.ds(o, 16)] = vz
              flats.append(vx * (_R * _R) + vy * _R + vz)
            idx_v[pl.ds((base + off) // 2, 16)] = (
                flats[0] | (flats[1] << 16))
            return 0

          lax.fori_loop(0, _CCH // 32, p1v_full, 0)
          pltpu.async_copy(vbuf.at[slot],
                           vox_hbm.at[b, :, pl.ds(base, _CCH)],
                           sems.at[6 + slot])

        @pl.when(g != 0)
        def _():
          def p1v_idx(i, _):
            off = i * 32
            flats = []
            for u in range(2):
              o = off + u * 16
              vx = axis_round(cbuf[slot, 0, pl.ds(o, 16)])
              vy = axis_round(cbuf[slot, 1, pl.ds(o, 16)])
              vz = axis_round(cbuf[slot, 2, pl.ds(o, 16)])
              flats.append(vx * (_R * _R) + vy * _R + vz)
            idx_v[pl.ds((base + off) // 2, 16)] = (
                flats[0] | (flats[1] << 16))
            return 0

          lax.fori_loop(0, _CCH // 32, p1v_idx, 0)

        nxt = base + 2 * _CCH

        @pl.when(nxt < _N)
        def _():
          pltpu.async_copy(c_hbm.at[b, :, pl.ds(nxt, _CCH)],
                           cbuf.at[slot], sems.at[4 + slot])
      return 0

    lax.fori_loop(0, _NCC // 2, p1_pair, 0)

    @pl.when(g == 0)
    def _():
      for slot in range(2):
        pltpu.make_async_copy(vbuf.at[slot],
                              vox_hbm.at[b, :, pl.ds(slot * _CCH, _CCH)],
                              sems.at[6 + slot]).wait()

    def zero_acc():
      def zloop(j, _):
        for u in range(8):
          acc_v[pl.ds((j * 8 + u) * 16, 16)] = zero16
        return 0

      lax.fori_loop(0, _NV // 128, zloop, 0)

    @pl.when(g == _GROUPS - 1)
    def _counts():
      zero_acc()

      def sloop(i, _):
        for u in range(4):
          o = (i * 4 + u) * 32
          w = idx_v[pl.ds(o // 2, 16)]
          iv0 = w & 0xFFFF
          iv1 = lax.shift_right_logical(w, 16)
          plsc.addupdate_scatter(acc_v, [iv0], ones16)
          plsc.addupdate_scatter(acc_v, [iv1], ones16)
        return 0

      lax.fori_loop(0, _N // 128, sloop, 0)

      def iloop(j, _):
        for u in range(4):
          o = (j * 4 + u) * 16
          v = acc_v[pl.ds(o, 16)]
          acc_v[pl.ds(o, 16)] = one / jnp.maximum(v, one)
        return 0

      lax.fori_loop(0, _NV // 64, iloop, 0)
      pltpu.sync_copy(acc_v, inv_s.at[bl])

    @pl.when(g != _GROUPS - 1)
    def _zero0():
      zero_acc()

    plsc.subcore_barrier()

    pltpu.sync_copy(inv_s.at[bl], inv_v)

    @pl.when(g == _GROUPS - 1)
    def _zero3():
      zero_acc()

    ch0 = g * _CPG
    for slot in range(2):
      pltpu.async_copy(f_hbm.at[b, ch0, pl.ds(slot * _FCH, _FCH)],
                       fbuf.at[slot], sems.at[slot])

    for cc in range(_CPG):
      ch = ch0 + cc

      def f_pair(kp, _):
        for slot in range(2):
          kc = kp * 2 + slot
          fbase = kc * _FCH
          pltpu.make_async_copy(f_hbm.at[b, ch, pl.ds(fbase, _FCH)],
                                fbuf.at[slot], sems.at[slot]).wait()

          def svec(i, _):
            for u in range(4):
              o = i * 128 + u * 32
              w = idx_v[pl.ds((fbase + o) // 2, 16)]
              iv0 = w & 0xFFFF
              iv1 = lax.shift_right_logical(w, 16)
              fv0 = fbuf[slot, pl.ds(o, 16)]
              fv1 = fbuf[slot, pl.ds(o + 16, 16)]
              plsc.addupdate_scatter(acc_v, [iv0], fv0)
              plsc.addupdate_scatter(acc_v, [iv1], fv1)
            return 0

          lax.fori_loop(0, _FCH // 128, svec, 0)

          nxt = fbase + 2 * _FCH

          @pl.when(nxt < _N)
          def _():
            pltpu.async_copy(f_hbm.at[b, ch, pl.ds(nxt, _FCH)],
                             fbuf.at[slot], sems.at[slot])
        return 0

      lax.fori_loop(0, _NCH // 2, f_pair, 0)

      if cc + 1 < _CPG:
        for slot in range(2):
          pltpu.async_copy(f_hbm.at[b, ch + 1, pl.ds(slot * _FCH, _FCH)],
                           fbuf.at[slot], sems.at[slot])

      def d_pair(mp, _):
        for slot in range(2):
          m = mp * 2 + slot
          dbase = m * _DCH

          @pl.when(mp >= 1)
          def _():
            pltpu.make_async_copy(dbuf.at[slot],
                                  out_hbm.at[b, ch, pl.ds(dbase, _DCH)],
                                  sems.at[2 + slot]).wait()

          def dvec(j, _):
            for u in range(8):
              o = (j * 8 + u) * 16
              a = acc_v[pl.ds(dbase + o, 16)]
              dbuf[slot, pl.ds(o, 16)] = a * inv_v[pl.ds(dbase + o, 16)]
              acc_v[pl.ds(dbase + o, 16)] = zero16
            return 0

          lax.fori_loop(0, _DCH // 128, dvec, 0)
          pltpu.async_copy(dbuf.at[slot],
                           out_hbm.at[b, ch, pl.ds(dbase, _DCH)],
                           sems.at[2 + slot])
        return 0

      lax.fori_loop(0, _NDC // 2, d_pair, 0)

      for slot in range(2):
        pltpu.make_async_copy(dbuf.at[slot],
                              out_hbm.at[b, ch, pl.ds(slot * _DCH, _DCH)],
                              sems.at[2 + slot]).wait()

  return vox_kernel


_vox_kernel = _make_sc_kernel()


@jax.jit
def kernel(features, coords):
  out_flat, vox_coords = _vox_kernel(features, coords)
  out = out_flat.reshape(_B, _FC, _R, _R, _R)
  return out, vox_coords
